# baseline (speedup 1.0000x reference)
import functools

import jax
import jax.numpy as jnp
from jax import lax
from jax.experimental import pallas as pl
from jax.experimental.pallas import tpu as pltpu
from jax.experimental.pallas import tpu_sc as plsc

N = 10000
E = 320000
D = 128
H = 2 * D

NC = 2
NS = 16
L = 16
NW = NC * NS
EPW = E // NW
K = 80
NCHUNK = EPW // K
RW = 80
NRCHUNK = N // RW
RT = -(-NRCHUNK // NS)
NBUF = 4


def _sc_body(src_hbm, dst_hbm, ea_hbm, nr_hbm, out_hbm,
             sidxs, didxs, msgs, acc, sema, semg, semsc):
    c = lax.axis_index("c")
    s = lax.axis_index("s")
    wid = s * NC + c
    zero = jnp.zeros((L,), jnp.float32)
    buf = msgs[0]

    def zrow(r, _):
        for j in range(D // L):
            buf[r, pl.ds(j * L, L)] = zero
        return ()
    lax.fori_loop(0, RW, zrow, ())
    for t in range(RT):
        cid = s + NS * t

        @pl.when(cid < NRCHUNK)
        def _():
            pltpu.sync_copy(buf, acc.at[pl.ds(cid * RW, RW), :])
    plsc.subcore_barrier()

    base = wid * NCHUNK * K

    def t0(ci, slot, drain):
        if drain:
            pltpu.make_async_copy(msgs[slot], acc.at[didxs[slot]],
                                  semsc.at[slot]).wait()
        pltpu.async_copy(ea_hbm.at[pl.ds(base + ci * K, K), :], msgs[slot],
                         sema.at[slot])
        pltpu.async_copy(src_hbm.at[wid, ci], sidxs[slot], sema.at[slot])
        pltpu.async_copy(dst_hbm.at[wid, ci], didxs[slot], sema.at[slot])

    def t1(slot):
        pltpu.make_async_copy(ea_hbm.at[pl.ds(base, K), :], msgs[slot],
                              sema.at[slot]).wait()
        pltpu.make_async_copy(src_hbm.at[wid, 0], sidxs[slot],
                              sema.at[slot]).wait()
        pltpu.make_async_copy(dst_hbm.at[wid, 0], didxs[slot],
                              sema.at[slot]).wait()
        pltpu.async_copy(nr_hbm.at[sidxs[slot]], msgs[slot], semg.at[slot],
                         add=True)

    def t2(slot):
        pltpu.make_async_copy(nr_hbm.at[sidxs[slot]], msgs[slot],
                              semg.at[slot]).wait()
        m = msgs[slot]

        def rrow(r, _):
            for j in range(D // L):
                v = m[r, pl.ds(j * L, L)]
                m[r, pl.ds(j * L, L)] = jnp.maximum(v, 0.0)
            return ()
        lax.fori_loop(0, K, rrow, ())
        pltpu.async_copy(m, acc.at[didxs[slot]], semsc.at[slot], add=True)

    t0(0, 0, False)
    t0(1, 1, False)
    t1(0)

    def step(ci, k, drain):
        @pl.when(ci + 2 < NCHUNK)
        def _():
            t0(ci + 2, (k + 2) % NBUF, drain)

        @pl.when(ci + 1 < NCHUNK)
        def _():
            t1((k + 1) % NBUF)
        t2(k)

    def body(i, _):
        for k in range(NBUF):
            ci = i * NBUF + k
            step(ci, k, True)
        return ()

    for k in range(min(NBUF, NCHUNK)):
        step(k, k, k >= 2)
    lax.fori_loop(1, NCHUNK // NBUF, body, ())
    for k in range(NCHUNK % NBUF):
        ci = (NCHUNK // NBUF) * NBUF + k
        step(ci, k, True)
    for k in range(min(NBUF, NCHUNK)):
        pltpu.make_async_copy(msgs[k], acc.at[didxs[k]], semsc.at[k]).wait()
    plsc.subcore_barrier()

    for t in range(RT):
        cid = s + NS * t

        @pl.when(cid < NRCHUNK)
        def _():
            pltpu.sync_copy(acc.at[pl.ds(cid * RW, RW), :], buf)
            pltpu.sync_copy(buf, out_hbm.at[c, pl.ds(cid * RW, RW), :])


def _sc_entry(src_hbm, dst_hbm, ea_hbm, nr_hbm, out_hbm,
              si0, si1, si2, si3, di0, di1, di2, di3,
              m0, m1, m2, m3, acc, sema, semg, semsc):
    _sc_body(src_hbm, dst_hbm, ea_hbm, nr_hbm, out_hbm,
             [si0, si1, si2, si3], [di0, di1, di2, di3],
             [m0, m1, m2, m3], acc, sema, semg, semsc)


@functools.cache
def _sc_segment():
    return pl.kernel(
        _sc_entry,
        out_type=jax.ShapeDtypeStruct((NC, N, D), jnp.float32),
        mesh=plsc.VectorSubcoreMesh(core_axis_name="c", subcore_axis_name="s",
                                    num_cores=NC, num_subcores=NS),
        scratch_types=(
            [pltpu.VMEM((K,), jnp.int32)] * NBUF
            + [pltpu.VMEM((K,), jnp.int32)] * NBUF
            + [pltpu.VMEM((K, D), jnp.float32)] * NBUF
            + [
                pltpu.VMEM_SHARED((N, D), jnp.float32),
                pltpu.SemaphoreType.DMA((NBUF,)),
                pltpu.SemaphoreType.DMA((NBUF,)),
                pltpu.SemaphoreType.DMA((NBUF,)),
            ]
        ),
    )


def _mlp_body(parts_ref, nr_ref, w1_ref, g1_ref, b1_ref, w2_ref, g2_ref,
              b2_ref, eps_ref, out_ref):
    h = parts_ref[0] + parts_ref[1] + (1.0 + eps_ref[0]) * nr_ref[...]
    y = jnp.dot(h, w1_ref[...], preferred_element_type=jnp.float32)
    mu = jnp.mean(y, axis=0, keepdims=True)
    var = jnp.mean((y - mu) ** 2, axis=0, keepdims=True)
    y = jnp.maximum((y - mu) * lax.rsqrt(var + 1e-5) * g1_ref[...]
                    + b1_ref[...], 0.0)
    z = jnp.dot(y, w2_ref[...], preferred_element_type=jnp.float32)
    mu2 = jnp.mean(z, axis=0, keepdims=True)
    var2 = jnp.mean((z - mu2) ** 2, axis=0, keepdims=True)
    out_ref[...] = jnp.maximum((z - mu2) * lax.rsqrt(var2 + 1e-5) * g2_ref[...]
                               + b2_ref[...], 0.0)


_mlp = pl.pallas_call(
    _mlp_body,
    out_shape=jax.ShapeDtypeStruct((N, D), jnp.float32),
    in_specs=[pl.BlockSpec(memory_space=pltpu.VMEM)] * 8
    + [pl.BlockSpec(memory_space=pltpu.SMEM)],
)


def kernel(node_rep, edge_index, edge_attr, W1, g1, b1, W2, g2, b2, epsilon):
    src = edge_index[0].reshape(NW, NCHUNK, K)
    dst = edge_index[1].reshape(NW, NCHUNK, K)
    parts = _sc_segment()(src, dst, edge_attr, node_rep)
    return _mlp(parts, node_rep, W1, g1.reshape(1, H), b1.reshape(1, H),
                W2, g2.reshape(1, D), b2.reshape(1, D), epsilon)


# speedup vs baseline: 1.0053x; 1.0053x over previous
import functools

import jax
import jax.numpy as jnp
from jax import lax
from jax.experimental import pallas as pl
from jax.experimental.pallas import tpu as pltpu
from jax.experimental.pallas import tpu_sc as plsc

N = 10000
E = 320000
D = 128
H = 2 * D

NC = 2
NS = 16
L = 16
NW = NC * NS
EPW = E // NW
K = 80
NCHUNK = EPW // K
RW = 80
NRCHUNK = N // RW
RT = -(-NRCHUNK // NS)
NBUF = 4


def _sc_body(src_hbm, dst_hbm, ea_hbm, nr_hbm, out_hbm,
             sidxs, didxs, msgs, acc, sema, semg, semsc):
    c = lax.axis_index("c")
    s = lax.axis_index("s")
    wid = s * NC + c
    zero = jnp.zeros((L,), jnp.float32)
    buf = msgs[0]

    def zrow(r, _):
        for j in range(D // L):
            buf[r, pl.ds(j * L, L)] = zero
        return ()
    lax.fori_loop(0, RW, zrow, ())
    for t in range(RT):
        cid = s + NS * t

        @pl.when(cid < NRCHUNK)
        def _():
            pltpu.async_copy(buf, acc.at[pl.ds(cid * RW, RW), :], sema.at[0])
    for t in range(RT):
        cid = s + NS * t

        @pl.when(cid < NRCHUNK)
        def _():
            pltpu.make_async_copy(buf, acc.at[pl.ds(0, RW), :],
                                  sema.at[0]).wait()
    plsc.subcore_barrier()

    base = wid * NCHUNK * K

    def t0(ci, slot, drain):
        if drain:
            pltpu.make_async_copy(msgs[slot], acc.at[didxs[slot]],
                                  semsc.at[slot]).wait()
        pltpu.async_copy(ea_hbm.at[pl.ds(base + ci * K, K), :], msgs[slot],
                         sema.at[slot])
        pltpu.async_copy(src_hbm.at[wid, ci], sidxs[slot], sema.at[slot])
        pltpu.async_copy(dst_hbm.at[wid, ci], didxs[slot], sema.at[slot])

    def t1(slot):
        pltpu.make_async_copy(ea_hbm.at[pl.ds(base, K), :], msgs[slot],
                              sema.at[slot]).wait()
        pltpu.make_async_copy(src_hbm.at[wid, 0], sidxs[slot],
                              sema.at[slot]).wait()
        pltpu.make_async_copy(dst_hbm.at[wid, 0], didxs[slot],
                              sema.at[slot]).wait()
        pltpu.async_copy(nr_hbm.at[sidxs[slot]], msgs[slot], semg.at[slot],
                         add=True)

    def t2(slot):
        pltpu.make_async_copy(nr_hbm.at[sidxs[slot]], msgs[slot],
                              semg.at[slot]).wait()
        m = msgs[slot]

        def rrow(r, _):
            for j in range(D // L):
                v = m[r, pl.ds(j * L, L)]
                m[r, pl.ds(j * L, L)] = jnp.maximum(v, 0.0)
            return ()
        lax.fori_loop(0, K, rrow, ())
        pltpu.async_copy(m, acc.at[didxs[slot]], semsc.at[slot], add=True)

    t0(0, 0, False)
    t0(1, 1, False)
    t1(0)

    def step(ci, k, drain):
        @pl.when(ci + 2 < NCHUNK)
        def _():
            t0(ci + 2, (k + 2) % NBUF, drain)

        @pl.when(ci + 1 < NCHUNK)
        def _():
            t1((k + 1) % NBUF)
        t2(k)

    def body(i, _):
        for k in range(NBUF):
            ci = i * NBUF + k
            step(ci, k, True)
        return ()

    for k in range(min(NBUF, NCHUNK)):
        step(k, k, k >= 2)
    lax.fori_loop(1, NCHUNK // NBUF, body, ())
    for k in range(NCHUNK % NBUF):
        ci = (NCHUNK // NBUF) * NBUF + k
        step(ci, k, True)
    for k in range(min(NBUF, NCHUNK)):
        pltpu.make_async_copy(msgs[k], acc.at[didxs[k]], semsc.at[k]).wait()
    plsc.subcore_barrier()

    for t in range(RT):
        cid = s + NS * t

        @pl.when(cid < NRCHUNK)
        def _():
            pltpu.async_copy(acc.at[pl.ds(cid * RW, RW), :],
                             out_hbm.at[c, pl.ds(cid * RW, RW), :],
                             sema.at[1])
    for t in range(RT):
        cid = s + NS * t

        @pl.when(cid < NRCHUNK)
        def _():
            pltpu.make_async_copy(acc.at[pl.ds(0, RW), :],
                                  out_hbm.at[c, pl.ds(0, RW), :],
                                  sema.at[1]).wait()


def _sc_entry(src_hbm, dst_hbm, ea_hbm, nr_hbm, out_hbm,
              si0, si1, si2, si3, di0, di1, di2, di3,
              m0, m1, m2, m3, acc, sema, semg, semsc):
    _sc_body(src_hbm, dst_hbm, ea_hbm, nr_hbm, out_hbm,
             [si0, si1, si2, si3], [di0, di1, di2, di3],
             [m0, m1, m2, m3], acc, sema, semg, semsc)


@functools.cache
def _sc_segment():
    return pl.kernel(
        _sc_entry,
        out_type=jax.ShapeDtypeStruct((NC, N, D), jnp.float32),
        mesh=plsc.VectorSubcoreMesh(core_axis_name="c", subcore_axis_name="s",
                                    num_cores=NC, num_subcores=NS),
        scratch_types=(
            [pltpu.VMEM((K,), jnp.int32)] * NBUF
            + [pltpu.VMEM((K,), jnp.int32)] * NBUF
            + [pltpu.VMEM((K, D), jnp.float32)] * NBUF
            + [
                pltpu.VMEM_SHARED((N, D), jnp.float32),
                pltpu.SemaphoreType.DMA((NBUF,)),
                pltpu.SemaphoreType.DMA((NBUF,)),
                pltpu.SemaphoreType.DMA((NBUF,)),
            ]
        ),
    )


def _mlp_body(parts_ref, nr_ref, w1_ref, g1_ref, b1_ref, w2_ref, g2_ref,
              b2_ref, eps_ref, out_ref):
    h = parts_ref[0] + parts_ref[1] + (1.0 + eps_ref[0]) * nr_ref[...]
    y = jnp.dot(h, w1_ref[...], preferred_element_type=jnp.float32)
    mu = jnp.mean(y, axis=0, keepdims=True)
    var = jnp.mean((y - mu) ** 2, axis=0, keepdims=True)
    y = jnp.maximum((y - mu) * lax.rsqrt(var + 1e-5) * g1_ref[...]
                    + b1_ref[...], 0.0)
    z = jnp.dot(y, w2_ref[...], preferred_element_type=jnp.float32)
    mu2 = jnp.mean(z, axis=0, keepdims=True)
    var2 = jnp.mean((z - mu2) ** 2, axis=0, keepdims=True)
    out_ref[...] = jnp.maximum((z - mu2) * lax.rsqrt(var2 + 1e-5) * g2_ref[...]
                               + b2_ref[...], 0.0)


_mlp = pl.pallas_call(
    _mlp_body,
    out_shape=jax.ShapeDtypeStruct((N, D), jnp.float32),
    in_specs=[pl.BlockSpec(memory_space=pltpu.VMEM)] * 8
    + [pl.BlockSpec(memory_space=pltpu.SMEM)],
)


def kernel(node_rep, edge_index, edge_attr, W1, g1, b1, W2, g2, b2, epsilon):
    src = edge_index[0].reshape(NW, NCHUNK, K)
    dst = edge_index[1].reshape(NW, NCHUNK, K)
    parts = _sc_segment()(src, dst, edge_attr, node_rep)
    return _mlp(parts, node_rep, W1, g1.reshape(1, H), b1.reshape(1, H),
                W2, g2.reshape(1, D), b2.reshape(1, D), epsilon)
